# Initial kernel scaffold; baseline (speedup 1.0000x reference)
import jax
import jax.numpy as jnp
from jax.experimental import pallas as pl


def kernel(x, Wq, bq, Wk, bk, Wv, bv, Wp, bp):
    raise NotImplementedError("write your pallas kernel here")


# speedup vs baseline: 2.3054x; 2.3054x over previous
import math

import jax
import jax.numpy as jnp
import numpy as np
from jax.experimental import pallas as pl
from jax.experimental.pallas import tpu as pltpu

_B, _N, _DIM, _H, _FACTOR = 4, 2048, 1024, 16, 5
_D = _DIM // _H
_U = min(_FACTOR * int(np.ceil(np.log(_N))), _N)
_SCALE = float(_D) ** -0.5
_QB = 256

_IDX = np.asarray(jax.random.randint(jax.random.key(42), (_N, _U), 0, _N))
_CNT_T = np.zeros((_N, _N), dtype=np.float32)
np.add.at(_CNT_T, (_IDX.reshape(-1), np.repeat(np.arange(_N), _U)), 1.0)


def _proj_kernel(x_ref, w_ref, b_ref, o_ref):
    o_ref[...] = jax.lax.dot_general(
        x_ref[...], w_ref[...], (((1,), (1,)), ((), ())),
        preferred_element_type=jnp.float32) + b_ref[...]


def _qkv_kernel(x_ref, wq_ref, wk_ref, wv_ref, bq_ref, bk_ref, bv_ref,
                q_ref, k_ref, v_ref):
    x = x_ref[...]
    nt = (((1,), (1,)), ((), ()))
    q_ref[...] = jax.lax.dot_general(
        x, wq_ref[...], nt, preferred_element_type=jnp.float32) + bq_ref[...]
    k_ref[...] = jax.lax.dot_general(
        x, wk_ref[...], nt, preferred_element_type=jnp.float32) + bk_ref[...]
    v_ref[...] = jax.lax.dot_general(
        x, wv_ref[...], nt, preferred_element_type=jnp.float32) + bv_ref[...]


def _attn_kernel(q_ref, k_ref, v_ref, cntT_ref, o_ref, oh_ref):
    q = q_ref[0, 0]
    k = k_ref[0, 0]
    v = v_ref[0, 0]
    nt = (((1,), (1,)), ((), ()))

    m_parts = []
    for i in range(_N // _QB):
        qb = q[i * _QB:(i + 1) * _QB]
        s_t = jax.lax.dot_general(k, qb, nt,
                                  preferred_element_type=jnp.float32)
        c = cntT_ref[:, i * _QB:(i + 1) * _QB]
        mx = jnp.max(jnp.where(c > 0.0, s_t, -1e30), axis=0, keepdims=True)
        ws = jnp.sum(s_t * c, axis=0, keepdims=True)
        m_parts.append(mx - ws * (1.0 / _N))
    m = jnp.concatenate(m_parts, axis=1)

    lane = jax.lax.broadcasted_iota(jnp.int32, (1, _N), 1)

    def body(j, m_cur):
        mval = jnp.max(m_cur)
        idx = jnp.min(jnp.where(m_cur == mval, lane, _N))
        oh_ref[pl.ds(j, 1), :] = (lane == idx).astype(jnp.float32)
        return jnp.where(lane == idx, -1e30, m_cur)

    jax.lax.fori_loop(0, _U, body, m)
    oh = oh_ref[...]

    qr = jnp.dot(oh, q, preferred_element_type=jnp.float32)
    sc = jax.lax.dot_general(qr, k, nt,
                             preferred_element_type=jnp.float32) * _SCALE
    sc = sc - jnp.max(sc, axis=1, keepdims=True)
    e = jnp.exp(sc)
    attn = e / jnp.sum(e, axis=1, keepdims=True)
    upd = jnp.dot(attn, v, preferred_element_type=jnp.float32)

    vmean = jnp.mean(v, axis=0, keepdims=True)
    o_ref[0, 0] = vmean + jax.lax.dot_general(
        oh, upd - vmean, (((0,), (0,)), ((), ())),
        preferred_element_type=jnp.float32)


def _matmul_bias(x2d, w, b, blk=512):
    n = x2d.shape[0]
    return pl.pallas_call(
        _proj_kernel,
        grid=(n // blk,),
        in_specs=[
            pl.BlockSpec((blk, _DIM), lambda i: (i, 0)),
            pl.BlockSpec((_DIM, _DIM), lambda i: (0, 0)),
            pl.BlockSpec((1, _DIM), lambda i: (0, 0)),
        ],
        out_specs=pl.BlockSpec((blk, _DIM), lambda i: (i, 0)),
        out_shape=jax.ShapeDtypeStruct((n, _DIM), jnp.float32),
    )(x2d, w, b.reshape(1, _DIM))


def kernel(x, Wq, bq, Wk, bk, Wv, bv, Wp, bp):
    Bx, Nx, C = x.shape
    x2d = x.reshape(Bx * Nx, C)
    blk = 512
    wspec = pl.BlockSpec((_DIM, _DIM), lambda i: (0, 0))
    bspec = pl.BlockSpec((1, _DIM), lambda i: (0, 0))
    rspec = pl.BlockSpec((blk, _DIM), lambda i: (i, 0))
    rshape = jax.ShapeDtypeStruct((Bx * Nx, _DIM), jnp.float32)
    q2d, k2d, v2d = pl.pallas_call(
        _qkv_kernel,
        grid=(Bx * Nx // blk,),
        in_specs=[rspec, wspec, wspec, wspec, bspec, bspec, bspec],
        out_specs=[rspec, rspec, rspec],
        out_shape=[rshape, rshape, rshape],
    )(x2d, Wq, Wk, Wv, bq.reshape(1, _DIM), bk.reshape(1, _DIM),
      bv.reshape(1, _DIM))

    q4 = q2d.reshape(Bx, Nx, _H, _D).transpose(0, 2, 1, 3)
    k4 = k2d.reshape(Bx, Nx, _H, _D).transpose(0, 2, 1, 3)
    v4 = v2d.reshape(Bx, Nx, _H, _D).transpose(0, 2, 1, 3)
    cntT = jnp.asarray(_CNT_T)

    hspec = pl.BlockSpec((1, 1, _N, _D), lambda b, h: (b, h, 0, 0))
    ctx = pl.pallas_call(
        _attn_kernel,
        grid=(Bx, _H),
        in_specs=[hspec, hspec, hspec,
                  pl.BlockSpec((_N, _N), lambda b, h: (0, 0))],
        out_specs=hspec,
        out_shape=jax.ShapeDtypeStruct((Bx, _H, Nx, _D), jnp.float32),
        scratch_shapes=[pltpu.VMEM((_U, _N), jnp.float32)],
    )(q4, k4, v4, cntT)

    ctx2d = ctx.transpose(0, 2, 1, 3).reshape(Bx * Nx, C)
    out = _matmul_bias(ctx2d, Wp, bp)
    return out.reshape(Bx, Nx, C)
